# baseline (speedup 1.0000x reference)
import functools

import jax
import jax.numpy as jnp
from jax import lax
from jax.experimental import pallas as pl
from jax.experimental.pallas import tpu as pltpu
from jax.experimental.pallas import tpu_sc as plsc

_MAX_VAL = 10
_EMB = 128
_BATCH = 16384

_NC = 2
_NS = 16
_NW = _NC * _NS
_BPW = _BATCH // _NW
_CH = 4
_CB = _BPW // _CH
_L = 16


def _body(values_hbm, table_hbm, out_hbm, vals_v, table_v, rows_v, sem):
    wid = lax.axis_index("s") * _NC + lax.axis_index("c")
    base = wid * _BPW
    pltpu.sync_copy(values_hbm.at[pl.ds(base, _BPW)], vals_v)
    pltpu.sync_copy(table_hbm, table_v)
    lane = lax.iota(jnp.int32, _L)

    @plsc.parallel_loop(0, _BPW // _L, carry=jnp.int32(0))
    def group(g, carry):
        src16 = (vals_v[pl.ds(g * _L, _L)] + _MAX_VAL) * _EMB
        dst16 = (g * _L + lane) * _EMB
        for cb in range(0, _EMB, 8):
            xs = [plsc.load_gather(table_v, [src16 + (cb + k)])
                  for k in range(8)]
            for k in range(8):
                plsc.store_scatter(rows_v, [dst16 + (cb + k)], xs[k])
        return carry
    pltpu.sync_copy(rows_v, out_hbm.at[pl.ds(base * _EMB, _BPW * _EMB)])


@jax.jit
def kernel(values, emb_table):
    run = pl.kernel(
        _body,
        mesh=plsc.VectorSubcoreMesh(core_axis_name="c", subcore_axis_name="s"),
        compiler_params=pltpu.CompilerParams(needs_layout_passes=False),
        out_type=jax.ShapeDtypeStruct((_BATCH * _EMB,), jnp.float32),
        scratch_types=[
            pltpu.VMEM((_BPW,), jnp.int32),
            pltpu.VMEM(((2 * _MAX_VAL + 1) * _EMB,), jnp.float32),
            pltpu.VMEM((_BPW * _EMB,), jnp.float32),
            pltpu.SemaphoreType.DMA,
        ],
    )
    return run(values, emb_table.reshape(-1)).reshape(_BATCH, _EMB)


# speedup vs baseline: 2.8270x; 2.8270x over previous
import jax
import jax.numpy as jnp
from jax import lax
from jax.experimental import pallas as pl
from jax.experimental.pallas import tpu as pltpu
from jax.experimental.pallas import tpu_sc as plsc

_MAX_VAL = 10
_EMB = 128
_BATCH = 16384
_NROWS = 2 * _MAX_VAL + 1

_NC = 2
_NS = 16
_NW = _NC * _NS
_BPW = _BATCH // _NW
_CH = 4
_CB = _BPW // _CH
_L = 16


def _body(values_hbm, table_hbm, out_hbm, vals_v, idx_v, table_sh, rows_v,
          gsem, wsem):
    wid = lax.axis_index("s") * _NC + lax.axis_index("c")
    base = wid * _BPW
    pltpu.sync_copy(values_hbm.at[pl.ds(base, _BPW)], vals_v)
    @pl.when(lax.axis_index("s") == 0)
    def _stage_table():
        pltpu.sync_copy(table_hbm, table_sh)
    plsc.subcore_barrier()
    for j in range(_CH):
        for k in range(_CB // _L):
            idx_v[j, pl.ds(k * _L, _L)] = (
                vals_v[pl.ds(j * _CB + k * _L, _L)] + _MAX_VAL
            )
    writes = []
    for j in range(_CH):
        pltpu.async_copy(
            table_sh.at[idx_v.at[j]], rows_v.at[pl.ds(j * _CB, _CB)], gsem
        ).wait()
        writes.append(
            pltpu.async_copy(
                rows_v.at[pl.ds(j * _CB, _CB)],
                out_hbm.at[pl.ds(base + j * _CB, _CB)],
                wsem,
            )
        )
    for w in writes:
        w.wait()


@jax.jit
def kernel(values, emb_table):
    run = pl.kernel(
        _body,
        mesh=plsc.VectorSubcoreMesh(core_axis_name="c", subcore_axis_name="s"),
        compiler_params=pltpu.CompilerParams(needs_layout_passes=False),
        out_type=jax.ShapeDtypeStruct((_BATCH, _EMB), jnp.float32),
        scratch_types=[
            pltpu.VMEM((_BPW,), jnp.int32),
            pltpu.VMEM((_CH, _CB), jnp.int32),
            pltpu.VMEM_SHARED((_NROWS, _EMB), jnp.float32),
            pltpu.VMEM((_BPW, _EMB), jnp.float32),
            pltpu.SemaphoreType.DMA,
            pltpu.SemaphoreType.DMA,
        ],
    )
    return run(values, emb_table)
